# baseline (speedup 1.0000x reference)
import functools

import jax
import jax.numpy as jnp
from jax import lax
from jax.experimental import pallas as pl
from jax.experimental.pallas import tpu as pltpu
from jax.experimental.pallas import tpu_sc as plsc

N = 10000
E = 320000
D = 128
NC = 2
NS = 16
C = 128
RPT = 624
TAIL0 = RPT * NS
TAIL = N - TAIL0
NBUF = 3
CHUNKS = 81
BODIES = CHUNKS // NBUF
EPW = CHUNKS * C
EPAD = NC * NS * EPW


def _sc_aggregate(h_pad, pack, zeros):
    mesh = plsc.VectorSubcoreMesh(core_axis_name="c", subcore_axis_name="s")

    @functools.partial(
        pl.kernel,
        mesh=mesh,
        out_type=jax.ShapeDtypeStruct((NC, N, D), jnp.float32),
        scratch_types=[
            [pltpu.VMEM((2, C), jnp.int32) for _ in range(NBUF)],
            [pltpu.VMEM((C, D), jnp.float32) for _ in range(NBUF)],
            pltpu.VMEM_SHARED((N, D), jnp.float32),
            [pltpu.SemaphoreType.DMA for _ in range(NBUF)],
            [pltpu.SemaphoreType.DMA for _ in range(NBUF)],
            [pltpu.SemaphoreType.DMA for _ in range(NBUF)],
        ],
    )
    def agg_kernel(h_hbm, pack_hbm, zeros_hbm, out_hbm,
                   idx_v, rows_v, acc_sh, sems_i, sems_g, sems_s):
        c = lax.axis_index("c")
        s = lax.axis_index("s")
        w = c * NS + s
        row0 = s * RPT

        @pl.when(c == 0)
        def _():
            pltpu.sync_copy(h_hbm.at[pl.ds(row0, RPT)],
                            acc_sh.at[pl.ds(row0, RPT)])

            @pl.when(s == NS - 1)
            def _():
                pltpu.sync_copy(h_hbm.at[pl.ds(TAIL0, TAIL)],
                                acc_sh.at[pl.ds(TAIL0, TAIL)])

        @pl.when(c != 0)
        def _():
            pltpu.sync_copy(zeros_hbm.at[pl.ds(row0, RPT)],
                            acc_sh.at[pl.ds(row0, RPT)])

            @pl.when(s == NS - 1)
            def _():
                pltpu.sync_copy(zeros_hbm.at[pl.ds(TAIL0, TAIL)],
                                acc_sh.at[pl.ds(TAIL0, TAIL)])

        plsc.subcore_barrier()

        def wait_idx(j):
            pltpu.make_async_copy(pack_hbm.at[0], idx_v[j], sems_i[j]).wait()

        def wait_gather(j):
            pltpu.make_async_copy(h_hbm.at[idx_v[j].at[0]], rows_v[j],
                                  sems_g[j]).wait()

        def wait_scatter(j):
            pltpu.make_async_copy(rows_v[j], acc_sh.at[idx_v[j].at[1]],
                                  sems_s[j]).wait()

        def fire_idx(j, r):
            pltpu.async_copy(pack_hbm.at[r], idx_v[j], sems_i[j])

        def fire_gather(j):
            pltpu.async_copy(h_hbm.at[idx_v[j].at[0]], rows_v[j], sems_g[j])

        def fire_scatter(j):
            pltpu.async_copy(rows_v[j], acc_sh.at[idx_v[j].at[1]], sems_s[j],
                             add=True)

        base = w * CHUNKS
        for j in range(NBUF):
            fire_idx(j, base + j)

        def body(p, carry):
            g0 = base + p * NBUF
            for j in range(NBUF):
                wait_idx(j)
                fire_gather(j)
            for j in range(NBUF):
                wait_gather(j)
                fire_scatter(j)
            for j in range(NBUF):
                wait_scatter(j)
                fire_idx(j, g0 + NBUF + j)
            return carry

        lax.fori_loop(0, BODIES, body, 0)
        for j in range(NBUF):
            wait_idx(j)

        plsc.subcore_barrier()
        pltpu.sync_copy(acc_sh.at[pl.ds(row0, RPT)],
                        out_hbm.at[c, pl.ds(row0, RPT)])

        @pl.when(s == NS - 1)
        def _():
            pltpu.sync_copy(acc_sh.at[pl.ds(TAIL0, TAIL)],
                            out_hbm.at[c, pl.ds(TAIL0, TAIL)])

    return agg_kernel(h_pad, pack, zeros)


def _tc_mlp(agg, W, b):
    d_out = W.shape[1]
    BR = 1000

    def mm_kernel(a_ref, w_ref, b_ref, o_ref):
        x = a_ref[0] + a_ref[1]
        o_ref[...] = jnp.dot(x, w_ref[...],
                             preferred_element_type=jnp.float32) + b_ref[...]

    return pl.pallas_call(
        mm_kernel,
        grid=(N // BR,),
        in_specs=[
            pl.BlockSpec((2, BR, D), lambda i: (0, i, 0)),
            pl.BlockSpec((D, d_out), lambda i: (0, 0)),
            pl.BlockSpec((1, d_out), lambda i: (0, 0)),
        ],
        out_specs=pl.BlockSpec((BR, d_out), lambda i: (i, 0)),
        out_shape=jax.ShapeDtypeStruct((N, d_out), jnp.float32),
    )(agg, W, b.reshape(1, d_out))


def kernel(features, edge_index, W_in, b_in, W_hid, b_hid, W_out, b_out):
    pad = EPAD + NBUF * C - E
    pad_iota = jnp.arange(pad, dtype=jnp.int32)
    src = jnp.concatenate([edge_index[0], N + (pad_iota % 8)])
    dst = jnp.concatenate([edge_index[1], pad_iota % N])
    pack = jnp.stack([src.reshape(EPAD // C + NBUF, C),
                      dst.reshape(EPAD // C + NBUF, C)], axis=1)
    zeros = jnp.zeros((N, D), jnp.float32)
    h = features
    for W, b in ((W_in, b_in), (W_hid, b_hid), (W_out, b_out)):
        h_pad = jnp.concatenate([h, jnp.zeros((8, D), jnp.float32)])
        agg = _sc_aggregate(h_pad, pack, zeros)
        h = _tc_mlp(agg, W, b)
    return h


# speedup vs baseline: 1.0880x; 1.0880x over previous
import functools

import jax
import jax.numpy as jnp
from jax import lax
from jax.experimental import pallas as pl
from jax.experimental.pallas import tpu as pltpu
from jax.experimental.pallas import tpu_sc as plsc

N = 10000
E = 320000
D = 128
NC = 2
NS = 16
C = 128
RPT = 624
TAIL0 = RPT * NS
TAIL = N - TAIL0
NBUF = 3
CHUNKS = 81
BODIES = CHUNKS // NBUF
EPW = CHUNKS * C
EPAD = NC * NS * EPW


def _sc_aggregate(h_pad, src, dst, zeros):
    mesh = plsc.VectorSubcoreMesh(core_axis_name="c", subcore_axis_name="s")

    @functools.partial(
        pl.kernel,
        mesh=mesh,
        out_type=jax.ShapeDtypeStruct((NC, N, D), jnp.float32),
        scratch_types=[
            [pltpu.VMEM((C,), jnp.int32) for _ in range(NBUF)],
            [pltpu.VMEM((C,), jnp.int32) for _ in range(NBUF)],
            [pltpu.VMEM((C, D), jnp.float32) for _ in range(NBUF)],
            pltpu.VMEM_SHARED((N, D), jnp.float32),
            [pltpu.SemaphoreType.DMA for _ in range(NBUF)],
            [pltpu.SemaphoreType.DMA for _ in range(NBUF)],
            [pltpu.SemaphoreType.DMA for _ in range(NBUF)],
            [pltpu.SemaphoreType.DMA for _ in range(NBUF)],
        ],
    )
    def agg_kernel(h_hbm, src_hbm, dst_hbm, zeros_hbm, out_hbm,
                   src_v, dst_v, rows_v, acc_sh,
                   sems_a, sems_d, sems_g, sems_s):
        c = lax.axis_index("c")
        s = lax.axis_index("s")
        w = c * NS + s
        row0 = s * RPT

        @pl.when(c == 0)
        def _():
            pltpu.sync_copy(h_hbm.at[pl.ds(row0, RPT)],
                            acc_sh.at[pl.ds(row0, RPT)])

            @pl.when(s == NS - 1)
            def _():
                pltpu.sync_copy(h_hbm.at[pl.ds(TAIL0, TAIL)],
                                acc_sh.at[pl.ds(TAIL0, TAIL)])

        @pl.when(c != 0)
        def _():
            pltpu.sync_copy(zeros_hbm.at[pl.ds(row0, RPT)],
                            acc_sh.at[pl.ds(row0, RPT)])

            @pl.when(s == NS - 1)
            def _():
                pltpu.sync_copy(zeros_hbm.at[pl.ds(TAIL0, TAIL)],
                                acc_sh.at[pl.ds(TAIL0, TAIL)])

        plsc.subcore_barrier()

        def fire_src(j, g):
            eb = pl.multiple_of(g * C, 8)
            pltpu.async_copy(src_hbm.at[pl.ds(eb, C)], src_v[j], sems_a[j])

        def fire_dst(j, g):
            eb = pl.multiple_of(g * C, 8)
            pltpu.async_copy(dst_hbm.at[pl.ds(eb, C)], dst_v[j], sems_d[j])

        def fire_gather(j):
            pltpu.async_copy(h_hbm.at[src_v[j]], rows_v[j], sems_g[j])

        def fire_scatter(j):
            pltpu.async_copy(rows_v[j], acc_sh.at[dst_v[j]], sems_s[j],
                             add=True)

        def wait_src(j):
            pltpu.make_async_copy(src_hbm.at[pl.ds(0, C)], src_v[j],
                                  sems_a[j]).wait()

        def wait_dst(j):
            pltpu.make_async_copy(dst_hbm.at[pl.ds(0, C)], dst_v[j],
                                  sems_d[j]).wait()

        def wait_gather(j):
            pltpu.make_async_copy(h_hbm.at[src_v[j]], rows_v[j],
                                  sems_g[j]).wait()

        def wait_scatter(j):
            pltpu.make_async_copy(rows_v[j], acc_sh.at[dst_v[j]],
                                  sems_s[j]).wait()

        base = w * CHUNKS
        for j in range(NBUF):
            fire_src(j, base + j)
            fire_dst(j, base + j)
        for j in range(NBUF):
            wait_src(j)
            fire_gather(j)

        def body(p, carry):
            g0 = base + p * NBUF
            for j in range(NBUF):
                wait_gather(j)
                wait_dst(j)
                fire_scatter(j)
                fire_src(j, g0 + NBUF + j)
            for j in range(NBUF):
                wait_scatter(j)
                fire_dst(j, g0 + NBUF + j)
                wait_src(j)
                fire_gather(j)
            return carry

        lax.fori_loop(0, BODIES, body, 0)
        for j in range(NBUF):
            wait_gather(j)
            wait_dst(j)

        plsc.subcore_barrier()
        pltpu.sync_copy(acc_sh.at[pl.ds(row0, RPT)],
                        out_hbm.at[c, pl.ds(row0, RPT)])

        @pl.when(s == NS - 1)
        def _():
            pltpu.sync_copy(acc_sh.at[pl.ds(TAIL0, TAIL)],
                            out_hbm.at[c, pl.ds(TAIL0, TAIL)])

    return agg_kernel(h_pad, src, dst, zeros)


def _tc_mlp(agg, W, b):
    d_out = W.shape[1]
    BR = 1000

    def mm_kernel(a_ref, w_ref, b_ref, o_ref):
        x = a_ref[0] + a_ref[1]
        o_ref[...] = jnp.dot(x, w_ref[...],
                             preferred_element_type=jnp.float32) + b_ref[...]

    return pl.pallas_call(
        mm_kernel,
        grid=(N // BR,),
        in_specs=[
            pl.BlockSpec((2, BR, D), lambda i: (0, i, 0)),
            pl.BlockSpec((D, d_out), lambda i: (0, 0)),
            pl.BlockSpec((1, d_out), lambda i: (0, 0)),
        ],
        out_specs=pl.BlockSpec((BR, d_out), lambda i: (i, 0)),
        out_shape=jax.ShapeDtypeStruct((N, d_out), jnp.float32),
    )(agg, W, b.reshape(1, d_out))


def kernel(features, edge_index, W_in, b_in, W_hid, b_hid, W_out, b_out):
    pad = EPAD + NBUF * C - E
    pad_iota = jnp.arange(pad, dtype=jnp.int32)
    src = jnp.concatenate([edge_index[0], N + (pad_iota % 8)])
    dst = jnp.concatenate([edge_index[1], pad_iota % N])
    zeros = jnp.zeros((N, D), jnp.float32)
    h = features
    for W, b in ((W_in, b_in), (W_hid, b_hid), (W_out, b_out)):
        h_pad = jnp.concatenate([h, jnp.zeros((8, D), jnp.float32)])
        agg = _sc_aggregate(h_pad, src, dst, zeros)
        h = _tc_mlp(agg, W, b)
    return h
